# baseline (device time: 33335 ns/iter reference)
import jax
import jax.numpy as jnp
from jax import lax
from jax.experimental import pallas as pl
from jax.experimental.pallas import tpu as pltpu

N_DEV = 4
B_LOC = 2
SQ = 128
SKV = 128
HQ = 16
HQ_GRP = 4
DH = 64
D_MODEL = 512
D_QKV = 256

BF16 = jnp.bfloat16
F32 = jnp.float32


def kernel(x, Wq, K_ext, V_ext, Wo):

    def body(x_ref, wq_ref, k_hbm, v_hbm, wo_ref, out_ref,
             wq16, wo16, wq_comm, wo_comm,
             k_vmem, v_vmem, k_sems, v_sems,
             wq_send, wq_recv, wo_send, wo_recv):
        my_pos = lax.axis_index("i")
        left = lax.rem(my_pos + N_DEV - 1, N_DEV)
        right = lax.rem(my_pos + 1, N_DEV)
        opp = lax.rem(my_pos + 2, N_DEV)

        wq16[...] = wq_ref[...].astype(BF16)
        wo16[...] = wo_ref[...].astype(BF16)

        kv_copies = []
        for b in range(B_LOC):
            gb = B_LOC * my_pos + b
            for h in range(HQ):
                ck = pltpu.make_async_copy(
                    k_hbm.at[gb, :, h, :], k_vmem.at[b, h], k_sems.at[b, h])
                cv = pltpu.make_async_copy(
                    v_hbm.at[gb, :, h, :], v_vmem.at[b, h], v_sems.at[b, h])
                ck.start()
                cv.start()
                kv_copies += [ck, cv]

        barrier_sem = pltpu.get_barrier_semaphore()
        for nbr in (left, right, opp):
            pl.semaphore_signal(barrier_sem, inc=1, device_id=(nbr,),
                                device_id_type=pl.DeviceIdType.MESH)
        pl.semaphore_wait(barrier_sem, 3)

        rdmas = []
        for slot, tgt in ((0, right), (1, left), (2, opp)):
            for src, comm, ssem, rsem in (
                (wq16, wq_comm, wq_send, wq_recv),
                (wo16, wo_comm, wo_send, wo_recv),
            ):
                r = pltpu.make_async_remote_copy(
                    src_ref=src, dst_ref=comm.at[slot],
                    send_sem=ssem.at[slot], recv_sem=rsem.at[slot],
                    device_id=(tgt,), device_id_type=pl.DeviceIdType.MESH)
                r.start()
                rdmas.append(r)

        for c in kv_copies:
            c.wait()

        xm = x_ref[...].reshape(B_LOC * SQ, D_MODEL).astype(BF16)

        def compute(g, wq, wo, first):
            qm = jnp.dot(xm, wq, preferred_element_type=F32).astype(BF16)
            ctxs = []
            for b in range(B_LOC):
                qb = qm[b * SQ:(b + 1) * SQ]
                for hh in range(HQ_GRP):
                    head = g * HQ_GRP + hh
                    q = qb[:, hh * DH:(hh + 1) * DH]
                    k = k_vmem[b, head].astype(BF16)
                    v = v_vmem[b, head].astype(BF16)
                    s = lax.dot_general(
                        q, k, (((1,), (1,)), ((), ())),
                        preferred_element_type=F32) * 0.125
                    m = jnp.max(s, axis=1, keepdims=True)
                    p = jnp.exp(s - m)
                    w = (p / jnp.sum(p, axis=1, keepdims=True)).astype(BF16)
                    ctxs.append(jnp.dot(w, v, preferred_element_type=F32))
            ctx = jnp.concatenate(
                [jnp.concatenate(ctxs[b * HQ_GRP:(b + 1) * HQ_GRP], axis=1)
                 for b in range(B_LOC)], axis=0).astype(BF16)
            contrib = jnp.dot(ctx, wo, preferred_element_type=F32)
            contrib = contrib.reshape(B_LOC, SQ, D_MODEL)
            if first:
                out_ref[...] = contrib
            else:
                out_ref[...] = out_ref[...] + contrib

        compute(my_pos, wq16[...], wo16[...], first=True)

        for slot, g in ((0, left), (1, right), (2, opp)):
            rdmas[2 * slot].wait_recv()
            rdmas[2 * slot + 1].wait_recv()
            compute(g, wq_comm[slot], wo_comm[slot], first=False)

        for r in rdmas:
            r.wait_send()

    return pl.pallas_call(
        body,
        out_shape=jax.ShapeDtypeStruct((B_LOC, SQ, D_MODEL), F32),
        in_specs=[
            pl.BlockSpec(memory_space=pltpu.VMEM),
            pl.BlockSpec(memory_space=pltpu.VMEM),
            pl.BlockSpec(memory_space=pl.ANY),
            pl.BlockSpec(memory_space=pl.ANY),
            pl.BlockSpec(memory_space=pltpu.VMEM),
        ],
        out_specs=pl.BlockSpec(memory_space=pltpu.VMEM),
        scratch_shapes=[
            pltpu.VMEM((D_MODEL, D_QKV), BF16),
            pltpu.VMEM((D_QKV, D_MODEL), BF16),
            pltpu.VMEM((3, D_MODEL, D_QKV), BF16),
            pltpu.VMEM((3, D_QKV, D_MODEL), BF16),
            pltpu.VMEM((B_LOC, HQ, SKV, DH), F32),
            pltpu.VMEM((B_LOC, HQ, SKV, DH), F32),
            pltpu.SemaphoreType.DMA((B_LOC, HQ)),
            pltpu.SemaphoreType.DMA((B_LOC, HQ)),
            pltpu.SemaphoreType.DMA((3,)),
            pltpu.SemaphoreType.DMA((3,)),
            pltpu.SemaphoreType.DMA((3,)),
            pltpu.SemaphoreType.DMA((3,)),
        ],
        compiler_params=pltpu.CompilerParams(collective_id=0),
    )(x, Wq, K_ext, V_ext, Wo)


# device time: 22806 ns/iter; 1.4617x vs baseline; 1.4617x over previous
import jax
import jax.numpy as jnp
from jax import lax
from jax.experimental import pallas as pl
from jax.experimental.pallas import tpu as pltpu

N_DEV = 4
B_LOC = 2
SQ = 128
SKV = 128
HQ = 16
HQ_GRP = 4
DH = 64
D_MODEL = 512
D_QKV = 256

BF16 = jnp.bfloat16
F32 = jnp.float32


def kernel(x, Wq, K_ext, V_ext, Wo):
    my = lax.axis_index("i")
    K_own = jnp.transpose(
        lax.dynamic_slice_in_dim(K_ext, B_LOC * my, B_LOC, axis=0), (0, 2, 1, 3)
    ).astype(BF16)
    V_own = jnp.transpose(
        lax.dynamic_slice_in_dim(V_ext, B_LOC * my, B_LOC, axis=0), (0, 2, 1, 3)
    ).astype(BF16)

    def body(x_ref, wq_ref, k_ref, v_ref, wo_ref, out_ref,
             wq16, wo16, wq_comm, wo_comm,
             wq_send, wq_recv, wo_send, wo_recv):
        my_pos = lax.axis_index("i")
        left = lax.rem(my_pos + N_DEV - 1, N_DEV)
        right = lax.rem(my_pos + 1, N_DEV)
        opp = lax.rem(my_pos + 2, N_DEV)

        wq16[...] = wq_ref[...].astype(BF16)
        wo16[...] = wo_ref[...].astype(BF16)

        barrier_sem = pltpu.get_barrier_semaphore()
        for nbr in (left, right, opp):
            pl.semaphore_signal(barrier_sem, inc=1, device_id=(nbr,),
                                device_id_type=pl.DeviceIdType.MESH)
        pl.semaphore_wait(barrier_sem, 3)

        rdmas = []
        for slot, tgt in ((0, right), (1, left), (2, opp)):
            for src, comm, ssem, rsem in (
                (wq16, wq_comm, wq_send, wq_recv),
                (wo16, wo_comm, wo_send, wo_recv),
            ):
                r = pltpu.make_async_remote_copy(
                    src_ref=src, dst_ref=comm.at[slot],
                    send_sem=ssem.at[slot], recv_sem=rsem.at[slot],
                    device_id=(tgt,), device_id_type=pl.DeviceIdType.MESH)
                r.start()
                rdmas.append(r)

        xm = x_ref[...].reshape(B_LOC * SQ, D_MODEL).astype(BF16)

        def compute(g, wq, wo, first):
            qm = jnp.dot(xm, wq, preferred_element_type=F32).astype(BF16)
            ctxs = []
            for b in range(B_LOC):
                qb = qm[b * SQ:(b + 1) * SQ]
                for hh in range(HQ_GRP):
                    head = g * HQ_GRP + hh
                    q = qb[:, hh * DH:(hh + 1) * DH]
                    k = k_ref[b, head]
                    v = v_ref[b, head]
                    s = lax.dot_general(
                        q, k, (((1,), (1,)), ((), ())),
                        preferred_element_type=F32) * 0.125
                    m = jnp.max(s, axis=1, keepdims=True)
                    p = jnp.exp(s - m)
                    w = (p / jnp.sum(p, axis=1, keepdims=True)).astype(BF16)
                    ctxs.append(jnp.dot(w, v, preferred_element_type=F32))
            ctx = jnp.concatenate(
                [jnp.concatenate(ctxs[b * HQ_GRP:(b + 1) * HQ_GRP], axis=1)
                 for b in range(B_LOC)], axis=0).astype(BF16)
            contrib = jnp.dot(ctx, wo, preferred_element_type=F32)
            contrib = contrib.reshape(B_LOC, SQ, D_MODEL)
            if first:
                out_ref[...] = contrib
            else:
                out_ref[...] = out_ref[...] + contrib

        compute(my_pos, wq16[...], wo16[...], first=True)

        for slot, g in ((0, left), (1, right), (2, opp)):
            rdmas[2 * slot].wait_recv()
            rdmas[2 * slot + 1].wait_recv()
            compute(g, wq_comm[slot], wo_comm[slot], first=False)

        for r in rdmas:
            r.wait_send()

    return pl.pallas_call(
        body,
        out_shape=jax.ShapeDtypeStruct((B_LOC, SQ, D_MODEL), F32),
        in_specs=[pl.BlockSpec(memory_space=pltpu.VMEM)] * 5,
        out_specs=pl.BlockSpec(memory_space=pltpu.VMEM),
        scratch_shapes=[
            pltpu.VMEM((D_MODEL, D_QKV), BF16),
            pltpu.VMEM((D_QKV, D_MODEL), BF16),
            pltpu.VMEM((3, D_MODEL, D_QKV), BF16),
            pltpu.VMEM((3, D_QKV, D_MODEL), BF16),
            pltpu.SemaphoreType.DMA((3,)),
            pltpu.SemaphoreType.DMA((3,)),
            pltpu.SemaphoreType.DMA((3,)),
            pltpu.SemaphoreType.DMA((3,)),
        ],
        compiler_params=pltpu.CompilerParams(collective_id=0),
    )(x, Wq, K_own, V_own, Wo)


# device time: 21745 ns/iter; 1.5330x vs baseline; 1.0488x over previous
import jax
import jax.numpy as jnp
from jax import lax
from jax.experimental import pallas as pl
from jax.experimental.pallas import tpu as pltpu

N_DEV = 4
B_LOC = 2
SQ = 128
SKV = 128
HQ = 16
HQ_GRP = 4
DH = 64
D_MODEL = 512
D_QKV = 256

BF16 = jnp.bfloat16
F32 = jnp.float32


def kernel(x, Wq, K_ext, V_ext, Wo):
    my = lax.axis_index("i")
    K_own = jnp.transpose(
        lax.dynamic_slice_in_dim(K_ext, B_LOC * my, B_LOC, axis=0), (0, 2, 1, 3)
    ).astype(BF16)
    V_own = jnp.transpose(
        lax.dynamic_slice_in_dim(V_ext, B_LOC * my, B_LOC, axis=0), (0, 2, 1, 3)
    ).astype(BF16)

    def body(x_ref, wq_ref, k_ref, v_ref, wo_ref, out_ref,
             wq16, wo16, wq_comm, wo_comm,
             wq_send, wq_recv, wo_send, wo_recv):
        my_pos = lax.axis_index("i")
        left = lax.rem(my_pos + N_DEV - 1, N_DEV)
        right = lax.rem(my_pos + 1, N_DEV)
        opp = lax.rem(my_pos + 2, N_DEV)

        wq16[...] = wq_ref[...].astype(BF16)
        wo16[...] = wo_ref[...].astype(BF16)

        barrier_sem = pltpu.get_barrier_semaphore()
        for nbr in (left, right, opp):
            pl.semaphore_signal(barrier_sem, inc=1, device_id=(nbr,),
                                device_id_type=pl.DeviceIdType.MESH)
        pl.semaphore_wait(barrier_sem, 3)

        rdmas = []
        for slot, tgt in ((0, right), (1, left), (2, opp)):
            for src, comm, ssem, rsem in (
                (wq16, wq_comm, wq_send, wq_recv),
                (wo16, wo_comm, wo_send, wo_recv),
            ):
                r = pltpu.make_async_remote_copy(
                    src_ref=src, dst_ref=comm.at[slot],
                    send_sem=ssem.at[slot], recv_sem=rsem.at[slot],
                    device_id=(tgt,), device_id_type=pl.DeviceIdType.MESH)
                r.start()
                rdmas.append(r)

        xm = x_ref[...].reshape(B_LOC * SQ, D_MODEL).astype(BF16)

        def compute(g, wq, wo):
            qm = (jnp.dot(xm, wq, preferred_element_type=F32)
                  * 0.125).astype(BF16)
            ctxs = []
            for b in range(B_LOC):
                qb = qm[b * SQ:(b + 1) * SQ]
                for hh in range(HQ_GRP):
                    head = g * HQ_GRP + hh
                    q = qb[:, hh * DH:(hh + 1) * DH]
                    k = k_ref[b, head]
                    v = v_ref[b, head]
                    s = lax.dot_general(
                        q, k, (((1,), (1,)), ((), ())),
                        preferred_element_type=F32)
                    p = jnp.exp(s)
                    r = 1.0 / jnp.sum(p, axis=1, keepdims=True)
                    ctx = jnp.dot(p.astype(BF16), v,
                                  preferred_element_type=F32) * r
                    ctxs.append(ctx)
            ctx = jnp.concatenate(
                [jnp.concatenate(ctxs[b * HQ_GRP:(b + 1) * HQ_GRP], axis=1)
                 for b in range(B_LOC)], axis=0).astype(BF16)
            return jnp.dot(ctx, wo, preferred_element_type=F32)

        acc = compute(my_pos, wq16[...], wo16[...])

        for slot, g in ((0, left), (1, right), (2, opp)):
            rdmas[2 * slot].wait_recv()
            rdmas[2 * slot + 1].wait_recv()
            acc = acc + compute(g, wq_comm[slot], wo_comm[slot])

        out_ref[...] = acc.reshape(B_LOC, SQ, D_MODEL)

        for r in rdmas:
            r.wait_send()

    return pl.pallas_call(
        body,
        out_shape=jax.ShapeDtypeStruct((B_LOC, SQ, D_MODEL), F32),
        in_specs=[pl.BlockSpec(memory_space=pltpu.VMEM)] * 5,
        out_specs=pl.BlockSpec(memory_space=pltpu.VMEM),
        scratch_shapes=[
            pltpu.VMEM((D_MODEL, D_QKV), BF16),
            pltpu.VMEM((D_QKV, D_MODEL), BF16),
            pltpu.VMEM((3, D_MODEL, D_QKV), BF16),
            pltpu.VMEM((3, D_QKV, D_MODEL), BF16),
            pltpu.SemaphoreType.DMA((3,)),
            pltpu.SemaphoreType.DMA((3,)),
            pltpu.SemaphoreType.DMA((3,)),
            pltpu.SemaphoreType.DMA((3,)),
        ],
        compiler_params=pltpu.CompilerParams(collective_id=0),
    )(x, Wq, K_own, V_own, Wo)


# device time: 20918 ns/iter; 1.5936x vs baseline; 1.0395x over previous
import jax
import jax.numpy as jnp
from jax import lax
from jax.experimental import pallas as pl
from jax.experimental.pallas import tpu as pltpu

N_DEV = 4
B_LOC = 2
SQ = 128
SKV = 128
HQ = 16
HQ_GRP = 4
DH = 64
D_MODEL = 512
D_QKV = 256

BF16 = jnp.bfloat16
F32 = jnp.float32


def kernel(x, Wq, K_ext, V_ext, Wo):
    my = lax.axis_index("i")
    K_own = jnp.transpose(
        lax.dynamic_slice_in_dim(K_ext, B_LOC * my, B_LOC, axis=0), (0, 2, 1, 3)
    ).astype(BF16)
    V_own = jnp.transpose(
        lax.dynamic_slice_in_dim(V_ext, B_LOC * my, B_LOC, axis=0), (0, 2, 1, 3)
    ).astype(BF16)

    def body(x_ref, wq_ref, k_ref, v_ref, wo_ref, out_ref,
             wq16, wo16, wq_comm, wo_comm,
             wq_send, wq_recv, wo_send, wo_recv):
        my_pos = lax.axis_index("i")
        left = lax.rem(my_pos + N_DEV - 1, N_DEV)
        right = lax.rem(my_pos + 1, N_DEV)
        opp = lax.rem(my_pos + 2, N_DEV)

        wq16[...] = wq_ref[...].astype(BF16)
        wo16[...] = wo_ref[...].astype(BF16)

        barrier_sem = pltpu.get_barrier_semaphore()
        for nbr in (left, right, opp):
            pl.semaphore_signal(barrier_sem, inc=1, device_id=(nbr,),
                                device_id_type=pl.DeviceIdType.MESH)
        pl.semaphore_wait(barrier_sem, 3)

        wq_rdmas, wo_rdmas = [], []
        for src, comm, ssem, rsem, out in (
            (wq16, wq_comm, wq_send, wq_recv, wq_rdmas),
            (wo16, wo_comm, wo_send, wo_recv, wo_rdmas),
        ):
            for slot, tgt in ((0, right), (1, left), (2, opp)):
                r = pltpu.make_async_remote_copy(
                    src_ref=src, dst_ref=comm.at[slot],
                    send_sem=ssem.at[slot], recv_sem=rsem.at[slot],
                    device_id=(tgt,), device_id_type=pl.DeviceIdType.MESH)
                r.start()
                out.append(r)

        xm = x_ref[...].reshape(B_LOC * SQ, D_MODEL).astype(BF16)

        def attention(g, wq):
            qm = (jnp.dot(xm, wq, preferred_element_type=F32)
                  * 0.125).astype(BF16)
            ctxs = []
            for b in range(B_LOC):
                qb = qm[b * SQ:(b + 1) * SQ]
                for hh in range(HQ_GRP):
                    head = g * HQ_GRP + hh
                    q = qb[:, hh * DH:(hh + 1) * DH]
                    k = k_ref[b, head]
                    v = v_ref[b, head]
                    s = lax.dot_general(
                        q, k, (((1,), (1,)), ((), ())),
                        preferred_element_type=F32)
                    p = jnp.exp(s)
                    r = 1.0 / jnp.sum(p, axis=1, keepdims=True)
                    ctx = jnp.dot(p.astype(BF16), v,
                                  preferred_element_type=F32) * r
                    ctxs.append(ctx)
            return jnp.concatenate(
                [jnp.concatenate(ctxs[b * HQ_GRP:(b + 1) * HQ_GRP], axis=1)
                 for b in range(B_LOC)], axis=0).astype(BF16)

        acc = jnp.dot(attention(my_pos, wq16[...]), wo16[...],
                      preferred_element_type=F32)

        for slot, g in ((0, left), (1, right), (2, opp)):
            wq_rdmas[slot].wait_recv()
            ctx = attention(g, wq_comm[slot])
            wo_rdmas[slot].wait_recv()
            acc = acc + jnp.dot(ctx, wo_comm[slot],
                                preferred_element_type=F32)

        out_ref[...] = acc.reshape(B_LOC, SQ, D_MODEL)

        for r in wq_rdmas + wo_rdmas:
            r.wait_send()

    return pl.pallas_call(
        body,
        out_shape=jax.ShapeDtypeStruct((B_LOC, SQ, D_MODEL), F32),
        in_specs=[pl.BlockSpec(memory_space=pltpu.VMEM)] * 5,
        out_specs=pl.BlockSpec(memory_space=pltpu.VMEM),
        scratch_shapes=[
            pltpu.VMEM((D_MODEL, D_QKV), BF16),
            pltpu.VMEM((D_QKV, D_MODEL), BF16),
            pltpu.VMEM((3, D_MODEL, D_QKV), BF16),
            pltpu.VMEM((3, D_QKV, D_MODEL), BF16),
            pltpu.SemaphoreType.DMA((3,)),
            pltpu.SemaphoreType.DMA((3,)),
            pltpu.SemaphoreType.DMA((3,)),
            pltpu.SemaphoreType.DMA((3,)),
        ],
        compiler_params=pltpu.CompilerParams(collective_id=0),
    )(x, Wq, K_own, V_own, Wo)


# device time: 20464 ns/iter; 1.6290x vs baseline; 1.0222x over previous
import jax
import jax.numpy as jnp
from jax import lax
from jax.experimental import pallas as pl
from jax.experimental.pallas import tpu as pltpu

N_DEV = 4
B_LOC = 2
SQ = 128
SKV = 128
HQ = 16
HQ_GRP = 4
DH = 64
D_MODEL = 512
D_QKV = 256

BF16 = jnp.bfloat16
F32 = jnp.float32


def kernel(x, Wq, K_ext, V_ext, Wo):
    my = lax.axis_index("i")
    K_own = jnp.transpose(
        lax.dynamic_slice_in_dim(K_ext, B_LOC * my, B_LOC, axis=0), (0, 2, 1, 3)
    ).astype(BF16)
    V_own = jnp.transpose(
        lax.dynamic_slice_in_dim(V_ext, B_LOC * my, B_LOC, axis=0), (0, 2, 1, 3)
    ).astype(BF16)

    def body(x_ref, wq_ref, k_ref, v_ref, wo_ref, out_ref,
             wq16, wo16, wq_comm, wo_comm,
             wq_send, wq_recv, wo_send, wo_recv):
        my_pos = lax.axis_index("i")
        left = lax.rem(my_pos + N_DEV - 1, N_DEV)
        right = lax.rem(my_pos + 1, N_DEV)
        opp = lax.rem(my_pos + 2, N_DEV)

        wq16[...] = wq_ref[...].astype(BF16)
        wo16[...] = wo_ref[...].astype(BF16)

        barrier_sem = pltpu.get_barrier_semaphore()
        for nbr in (left, right, opp):
            pl.semaphore_signal(barrier_sem, inc=1, device_id=(nbr,),
                                device_id_type=pl.DeviceIdType.MESH)
        pl.semaphore_wait(barrier_sem, 3)

        wq_rdmas, wo_rdmas = [], []
        for src, comm, ssem, rsem, out in (
            (wq16, wq_comm, wq_send, wq_recv, wq_rdmas),
            (wo16, wo_comm, wo_send, wo_recv, wo_rdmas),
        ):
            for slot, tgt in ((0, right), (1, left), (2, opp)):
                r = pltpu.make_async_remote_copy(
                    src_ref=src, dst_ref=comm.at[slot],
                    send_sem=ssem.at[slot], recv_sem=rsem.at[slot],
                    device_id=(tgt,), device_id_type=pl.DeviceIdType.MESH)
                r.start()
                out.append(r)

        xm = x_ref[...].reshape(B_LOC * SQ, D_MODEL).astype(BF16)

        def attention(g, wq):
            qm = (jnp.dot(xm, wq, preferred_element_type=F32)
                  * 0.125).astype(BF16)
            g4 = g * HQ_GRP
            out = []
            for b in range(B_LOC):
                qb = qm[b * SQ:(b + 1) * SQ].reshape(SQ, HQ_GRP, DH)
                kb = k_ref[b, pl.ds(g4, HQ_GRP)]
                vb = v_ref[b, pl.ds(g4, HQ_GRP)]
                s4 = lax.dot_general(
                    qb, kb, (((2,), (2,)), ((1,), (0,))),
                    preferred_element_type=F32)
                p4 = jnp.exp(s4)
                r4 = 1.0 / jnp.sum(p4, axis=2, keepdims=True)
                c4 = lax.dot_general(
                    p4.astype(BF16), vb, (((2,), (1,)), ((0,), (0,))),
                    preferred_element_type=F32) * r4
                out.append(c4.astype(BF16))
            return out

        def project(cs, wo):
            ctx = jnp.concatenate(
                [jnp.concatenate([cb[h] for h in range(HQ_GRP)], axis=1)
                 for cb in cs], axis=0)
            return jnp.dot(ctx, wo, preferred_element_type=F32)

        acc = project(attention(my_pos, wq16[...]), wo16[...])

        for slot, g in ((0, left), (1, right), (2, opp)):
            wq_rdmas[slot].wait_recv()
            cs = attention(g, wq_comm[slot])
            wo_rdmas[slot].wait_recv()
            acc = acc + project(cs, wo_comm[slot])

        out_ref[...] = acc.reshape(B_LOC, SQ, D_MODEL)

        for r in wq_rdmas + wo_rdmas:
            r.wait_send()

    return pl.pallas_call(
        body,
        out_shape=jax.ShapeDtypeStruct((B_LOC, SQ, D_MODEL), F32),
        in_specs=[pl.BlockSpec(memory_space=pltpu.VMEM)] * 5,
        out_specs=pl.BlockSpec(memory_space=pltpu.VMEM),
        scratch_shapes=[
            pltpu.VMEM((D_MODEL, D_QKV), BF16),
            pltpu.VMEM((D_QKV, D_MODEL), BF16),
            pltpu.VMEM((3, D_MODEL, D_QKV), BF16),
            pltpu.VMEM((3, D_QKV, D_MODEL), BF16),
            pltpu.SemaphoreType.DMA((3,)),
            pltpu.SemaphoreType.DMA((3,)),
            pltpu.SemaphoreType.DMA((3,)),
            pltpu.SemaphoreType.DMA((3,)),
        ],
        compiler_params=pltpu.CompilerParams(collective_id=0),
    )(x, Wq, K_own, V_own, Wo)
